# Initial kernel scaffold; baseline (speedup 1.0000x reference)
import functools

import jax
import jax.numpy as jnp
from jax import lax
from jax.experimental import pallas as pl
from jax.experimental.pallas import tpu as pltpu
from jax.experimental.pallas import tpu_sc as plsc

N = 10000
D = 128
E = 320000
OUT = 64
NC = 2
NS = 16
NW = NC * NS
NP = 10240
RPT = NP // NS
EPT = E // NW
K = 128
NCH = -(-EPT // K)

_mesh = plsc.VectorSubcoreMesh(core_axis_name="c", subcore_axis_name="s")


@functools.partial(
    pl.kernel,
    mesh=_mesh,
    out_type=jax.ShapeDtypeStruct((NC, NP, 16), jnp.float32),
    scratch_types=[
        pltpu.VMEM((NCH, K), jnp.int32),
        pltpu.VMEM((K, 16), jnp.float32),
        pltpu.VMEM_SHARED((NP, 16), jnp.float32),
    ],
)
def _sc_degree(dst_hbm, ones_hbm, zeros_hbm, out_hbm, dst_v, ones_v, dacc):
    cid = lax.axis_index("c")
    sid = lax.axis_index("s")
    wid = sid * NC + cid
    pltpu.sync_copy(dst_hbm.at[wid], dst_v)
    pltpu.sync_copy(ones_hbm, ones_v)
    pltpu.sync_copy(zeros_hbm, dacc.at[pl.ds(sid * RPT, RPT)])
    plsc.subcore_barrier()

    def body(j, c):
        pltpu.sync_copy(ones_v, dacc.at[dst_v.at[j]], add=True)
        return c

    lax.fori_loop(0, NCH, body, 0)
    plsc.subcore_barrier()
    pltpu.sync_copy(dacc.at[pl.ds(sid * RPT, RPT)],
                    out_hbm.at[cid, pl.ds(sid * RPT, RPT)])


@functools.partial(
    pl.kernel,
    mesh=_mesh,
    out_type=jax.ShapeDtypeStruct((NC, NP, D), jnp.float32),
    scratch_types=[
        pltpu.VMEM((NCH, K), jnp.int32),
        pltpu.VMEM((NCH, K), jnp.int32),
        pltpu.VMEM((K, D), jnp.float32),
        pltpu.VMEM_SHARED((NP, D), jnp.float32),
        pltpu.SemaphoreType.DMA,
    ],
)
def _sc_gather_scatter(hs_hbm, src_hbm, dst_hbm, zeros_hbm, out_hbm,
                       src_v, dst_v, buf, acc, sem):
    cid = lax.axis_index("c")
    sid = lax.axis_index("s")
    wid = sid * NC + cid
    pltpu.sync_copy(src_hbm.at[wid], src_v)
    pltpu.sync_copy(dst_hbm.at[wid], dst_v)
    pltpu.sync_copy(zeros_hbm, acc.at[pl.ds(sid * RPT, RPT)])
    plsc.subcore_barrier()

    def body(j, c):
        pltpu.async_copy(hs_hbm.at[src_v.at[j]], buf, sem).wait()
        pltpu.sync_copy(buf, acc.at[dst_v.at[j]], add=True)
        return c

    lax.fori_loop(0, NCH, body, 0)
    plsc.subcore_barrier()
    pltpu.sync_copy(acc.at[pl.ds(sid * RPT, RPT)],
                    out_hbm.at[cid, pl.ds(sid * RPT, RPT)])


def _dinv_full(degs_ref):
    deg = degs_ref[0, :, 0:1] + degs_ref[1, :, 0:1] + 1.0
    return lax.rsqrt(deg)


def _tc_first_body(x_ref, w_ref, degs_ref, hs_ref):
    dinv = _dinv_full(degs_ref)
    h = jnp.dot(x_ref[...], w_ref[...], preferred_element_type=jnp.float32)
    hs_ref[0:N, :] = h * dinv[0:N]
    hs_ref[N:NP, :] = jnp.zeros((NP - N, D), jnp.float32)


def _graphnorm_relu(accs_ref, hs_ref, degs_ref, b_ref, g_ref, be_ref, a_ref):
    dinv = _dinv_full(degs_ref)[0:N]
    asum = accs_ref[0, 0:N, :] + accs_ref[1, 0:N, :] + hs_ref[0:N, :]
    z = dinv * asum + b_ref[...]
    m = jnp.mean(z, axis=0, keepdims=True)
    sub = z - a_ref[...] * m
    var = jnp.mean(sub * sub, axis=0, keepdims=True)
    y = g_ref[...] * sub / jnp.sqrt(var + 1e-5) + be_ref[...]
    return jnp.maximum(y, 0.0), dinv


def _tc_mid_body(accs_ref, hs_ref, degs_ref, wn_ref, b_ref, g_ref, be_ref,
                 a_ref, out_ref):
    y, dinv = _graphnorm_relu(accs_ref, hs_ref, degs_ref, b_ref, g_ref,
                              be_ref, a_ref)
    hn = jnp.dot(y, wn_ref[...], preferred_element_type=jnp.float32)
    out_ref[0:N, :] = hn * dinv
    out_ref[N:NP, :] = jnp.zeros((NP - N, D), jnp.float32)


def _tc_last_body(accs_ref, hs_ref, degs_ref, wl_ref, b_ref, g_ref, be_ref,
                  a_ref, bl_ref, out_ref):
    y, _ = _graphnorm_relu(accs_ref, hs_ref, degs_ref, b_ref, g_ref,
                           be_ref, a_ref)
    pooled = jnp.mean(y, axis=0, keepdims=True)
    out_ref[...] = (jnp.dot(pooled, wl_ref[...],
                            preferred_element_type=jnp.float32) + bl_ref[...])


_tc_first = pl.pallas_call(
    _tc_first_body, out_shape=jax.ShapeDtypeStruct((NP, D), jnp.float32))
_tc_mid = pl.pallas_call(
    _tc_mid_body, out_shape=jax.ShapeDtypeStruct((NP, D), jnp.float32))
_tc_last = pl.pallas_call(
    _tc_last_body, out_shape=jax.ShapeDtypeStruct((1, OUT), jnp.float32))


def kernel(x, edge_index, W1, b1, g1, be1, a1, W2, b2, g2, be2, a2,
           W3, b3, g3, be3, a3, W4, b4, g4, be4, a4, Wl, bl):
    pad = NCH * K - EPT
    src_t = jnp.pad(edge_index[0].reshape(NW, EPT),
                    ((0, 0), (0, pad))).reshape(NW, NCH, K)
    dst_t = jnp.pad(edge_index[1].reshape(NW, EPT), ((0, 0), (0, pad)),
                    constant_values=NP - 1).reshape(NW, NCH, K)
    ones16 = jnp.ones((K, 16), jnp.float32)
    zeros16 = jnp.zeros((RPT, 16), jnp.float32)
    zerosD = jnp.zeros((RPT, D), jnp.float32)

    degs = _sc_degree(dst_t, ones16, zeros16)

    hs = _tc_first(x, W1, degs)
    layer = [(b1, g1, be1, a1), (b2, g2, be2, a2),
             (b3, g3, be3, a3), (b4, g4, be4, a4)]
    wnext = [W2, W3, W4]
    for i in range(3):
        accs = _sc_gather_scatter(hs, src_t, dst_t, zerosD)
        b, g, be, a = layer[i]
        hs = _tc_mid(accs, hs, degs, wnext[i], b.reshape(1, D),
                     g.reshape(1, D), be.reshape(1, D), a.reshape(1, D))
    accs = _sc_gather_scatter(hs, src_t, dst_t, zerosD)
    b, g, be, a = layer[3]
    return _tc_last(accs, hs, degs, Wl, b.reshape(1, D), g.reshape(1, D),
                    be.reshape(1, D), a.reshape(1, D), bl.reshape(1, OUT))


# speedup vs baseline: 10.4596x; 10.4596x over previous
import functools

import jax
import jax.numpy as jnp
from jax import lax
from jax.experimental import pallas as pl
from jax.experimental.pallas import tpu as pltpu
from jax.experimental.pallas import tpu_sc as plsc

N = 10000
D = 128
E = 320000
OUT = 64
NC = 2
NS = 16
NW = NC * NS
NP = 10240
RPT = NP // NS
EPT = E // NW
K = 128
NCH = -(-EPT // K)

_mesh = plsc.VectorSubcoreMesh(core_axis_name="c", subcore_axis_name="s")


@functools.partial(
    pl.kernel,
    mesh=_mesh,
    out_type=jax.ShapeDtypeStruct((NC, NP, D), jnp.float32),
    scratch_types=[
        pltpu.VMEM((NCH, K), jnp.int32),
        pltpu.VMEM((K, D), jnp.float32),
        pltpu.VMEM_SHARED((NP, D), jnp.float32),
    ],
)
def _sc_degree(dst_hbm, ones_hbm, zeros_hbm, out_hbm, dst_v, ones_v, dacc):
    cid = lax.axis_index("c")
    sid = lax.axis_index("s")
    wid = sid * NC + cid
    pltpu.sync_copy(dst_hbm.at[wid], dst_v)
    pltpu.sync_copy(ones_hbm, ones_v)
    pltpu.sync_copy(zeros_hbm, dacc.at[pl.ds(sid * RPT, RPT)])
    plsc.subcore_barrier()

    def body(j, c):
        pltpu.sync_copy(ones_v, dacc.at[dst_v.at[j]], add=True)
        return c

    lax.fori_loop(0, NCH, body, 0)
    plsc.subcore_barrier()
    pltpu.sync_copy(dacc.at[pl.ds(sid * RPT, RPT)],
                    out_hbm.at[cid, pl.ds(sid * RPT, RPT)])


@functools.partial(
    pl.kernel,
    mesh=_mesh,
    out_type=jax.ShapeDtypeStruct((NC, NP, D), jnp.float32),
    scratch_types=[
        pltpu.VMEM((NCH, K), jnp.int32),
        pltpu.VMEM((NCH, K), jnp.int32),
        pltpu.VMEM((K, D), jnp.float32),
        pltpu.VMEM_SHARED((NP, D), jnp.float32),
        pltpu.SemaphoreType.DMA,
    ],
)
def _sc_gather_scatter(hs_hbm, src_hbm, dst_hbm, zeros_hbm, out_hbm,
                       src_v, dst_v, buf, acc, sem):
    cid = lax.axis_index("c")
    sid = lax.axis_index("s")
    wid = sid * NC + cid
    pltpu.sync_copy(src_hbm.at[wid], src_v)
    pltpu.sync_copy(dst_hbm.at[wid], dst_v)
    pltpu.sync_copy(zeros_hbm, acc.at[pl.ds(sid * RPT, RPT)])
    plsc.subcore_barrier()

    def body(j, c):
        pltpu.async_copy(hs_hbm.at[src_v.at[j]], buf, sem).wait()
        pltpu.sync_copy(buf, acc.at[dst_v.at[j]], add=True)
        return c

    lax.fori_loop(0, NCH, body, 0)
    plsc.subcore_barrier()
    pltpu.sync_copy(acc.at[pl.ds(sid * RPT, RPT)],
                    out_hbm.at[cid, pl.ds(sid * RPT, RPT)])


def _tc_first_body(x_ref, w_ref, degs_ref, hs_ref, dinv_ref):
    deg = degs_ref[0, :, 0:1] + degs_ref[1, :, 0:1] + 1.0
    dinv = lax.rsqrt(deg)
    dinv_ref[...] = dinv
    h = jnp.dot(x_ref[...], w_ref[...], preferred_element_type=jnp.float32)
    hs_ref[0:N, :] = h * dinv[0:N]
    hs_ref[N:NP, :] = jnp.zeros((NP - N, D), jnp.float32)


def _graphnorm_relu(accs_ref, hs_ref, dinv_ref, b_ref, g_ref, be_ref, a_ref):
    dinv = dinv_ref[0:N]
    asum = accs_ref[0, 0:N, :] + accs_ref[1, 0:N, :] + hs_ref[0:N, :]
    z = dinv * asum + b_ref[...]
    m = jnp.mean(z, axis=0, keepdims=True)
    sub = z - a_ref[...] * m
    var = jnp.mean(sub * sub, axis=0, keepdims=True)
    y = g_ref[...] * sub / jnp.sqrt(var + 1e-5) + be_ref[...]
    return jnp.maximum(y, 0.0), dinv


def _tc_mid_body(accs_ref, hs_ref, dinv_ref, wn_ref, b_ref, g_ref, be_ref,
                 a_ref, out_ref):
    y, dinv = _graphnorm_relu(accs_ref, hs_ref, dinv_ref, b_ref, g_ref,
                              be_ref, a_ref)
    hn = jnp.dot(y, wn_ref[...], preferred_element_type=jnp.float32)
    out_ref[0:N, :] = hn * dinv
    out_ref[N:NP, :] = jnp.zeros((NP - N, D), jnp.float32)


def _tc_last_body(accs_ref, hs_ref, dinv_ref, wl_ref, b_ref, g_ref, be_ref,
                  a_ref, bl_ref, out_ref):
    y, _ = _graphnorm_relu(accs_ref, hs_ref, dinv_ref, b_ref, g_ref,
                           be_ref, a_ref)
    pooled = jnp.mean(y, axis=0, keepdims=True)
    out_ref[...] = (jnp.dot(pooled, wl_ref[...],
                            preferred_element_type=jnp.float32) + bl_ref[...])


_tc_first = pl.pallas_call(
    _tc_first_body,
    out_shape=(jax.ShapeDtypeStruct((NP, D), jnp.float32),
               jax.ShapeDtypeStruct((NP, 1), jnp.float32)))
_tc_mid = pl.pallas_call(
    _tc_mid_body, out_shape=jax.ShapeDtypeStruct((NP, D), jnp.float32))
_tc_last = pl.pallas_call(
    _tc_last_body, out_shape=jax.ShapeDtypeStruct((1, OUT), jnp.float32))


def kernel(x, edge_index, W1, b1, g1, be1, a1, W2, b2, g2, be2, a2,
           W3, b3, g3, be3, a3, W4, b4, g4, be4, a4, Wl, bl):
    pad = NCH * K - EPT
    src_t = jnp.pad(edge_index[0].reshape(NW, EPT),
                    ((0, 0), (0, pad))).reshape(NW, NCH, K)
    dst_t = jnp.pad(edge_index[1].reshape(NW, EPT), ((0, 0), (0, pad)),
                    constant_values=NP - 1).reshape(NW, NCH, K)
    onesD = jnp.ones((K, D), jnp.float32)
    zerosD = jnp.zeros((RPT, D), jnp.float32)

    degs = _sc_degree(dst_t, onesD, zerosD)

    hs, dinv = _tc_first(x, W1, degs)
    layer = [(b1, g1, be1, a1), (b2, g2, be2, a2),
             (b3, g3, be3, a3), (b4, g4, be4, a4)]
    wnext = [W2, W3, W4]
    for i in range(3):
        accs = _sc_gather_scatter(hs, src_t, dst_t, zerosD)
        b, g, be, a = layer[i]
        hs = _tc_mid(accs, hs, dinv, wnext[i], b.reshape(1, D),
                     g.reshape(1, D), be.reshape(1, D), a.reshape(1, D))
    accs = _sc_gather_scatter(hs, src_t, dst_t, zerosD)
    b, g, be, a = layer[3]
    return _tc_last(accs, hs, dinv, Wl, b.reshape(1, D), g.reshape(1, D),
                    be.reshape(1, D), a.reshape(1, D), bl.reshape(1, OUT))
